# baseline (speedup 1.0000x reference)
import functools

import jax
import jax.numpy as jnp
from jax import lax
from jax.experimental import pallas as pl
from jax.experimental.pallas import tpu as pltpu
from jax.experimental.pallas import tpu_sc as plsc

_BATCH = 16384
_HIST = 50
_HPAD = 56
_EMBED = 64

_info = plsc.get_sparse_core_info()
_NC, _NS = _info.num_cores, _info.num_subcores
_NW = _NC * _NS
_NBATCH = _BATCH // _NW
_CB = 4
_NCHUNK = _NBATCH // _CB
_NB = 4
_K = _NB // 2

_mesh = plsc.VectorSubcoreMesh(core_axis_name="c", subcore_axis_name="s")


@functools.partial(
    pl.kernel,
    mesh=_mesh,
    out_type=jax.ShapeDtypeStruct((_BATCH, _HIST, _EMBED), jnp.float32),
    scratch_types=[
        pltpu.VMEM((_NBATCH, _HPAD), jnp.int32),
    ]
    + [pltpu.VMEM((_CB, _HPAD, _EMBED), jnp.float32) for _ in range(_NB)]
    + [pltpu.SemaphoreType.DMA for _ in range(2 * _NB)],
    compiler_params=pltpu.CompilerParams(use_tc_tiling_on_sc=False),
)
def _gather_sc(idx_hbm, table_hbm, out_hbm, idx_v, *bufs_and_sems):
    rows = bufs_and_sems[:_NB]
    gsems = bufs_and_sems[_NB : 2 * _NB]
    ssems = bufs_and_sems[2 * _NB :]
    wid = lax.axis_index("s") * _NC + lax.axis_index("c")
    b0 = wid * _NBATCH
    pltpu.sync_copy(idx_hbm.at[wid], idx_v)

    def gather_start(c, b):
        for g in range(_CB):
            pltpu.async_copy(
                table_hbm.at[idx_v.at[c * _CB + g]], rows[b].at[g], gsems[b]
            )

    def gather_wait(c, b):
        for g in range(_CB):
            pltpu.make_async_copy(
                table_hbm.at[idx_v.at[c * _CB + g]], rows[b].at[g], gsems[b]
            ).wait()

    def store_start(c, b):
        pltpu.async_copy(
            rows[b].at[:, pl.ds(0, _HIST), :],
            out_hbm.at[pl.ds(b0 + c * _CB, _CB)],
            ssems[b],
        )

    def store_wait(c, b):
        pltpu.make_async_copy(
            rows[b].at[:, pl.ds(0, _HIST), :],
            out_hbm.at[pl.ds(b0 + c * _CB, _CB)],
            ssems[b],
        ).wait()

    for b in range(_K):
        gather_start(b, b)

    def body(c0):
        for b in range(_NB):
            c = c0 + b
            gather_wait(c, b)
            store_start(c, b)

            bk = (b - _K) % _NB

            @pl.when(c >= _K)
            def _():
                store_wait(c - _K, bk)

            @pl.when(c + _K < _NCHUNK)
            def _():
                gather_start(c + _K, bk)

    pl.loop(0, _NCHUNK, step=_NB)(body)

    for c in range(_NCHUNK - _K, _NCHUNK):
        store_wait(c, c % _NB)


def kernel(indices, table):
    idx = indices.astype(jnp.int32).reshape(_NW, _NBATCH, _HIST)
    pad = jnp.broadcast_to(idx[:, :, :1], (_NW, _NBATCH, _HPAD - _HIST))
    idx = jnp.concatenate([idx, pad], axis=-1)
    return _gather_sc(idx, table)


# speedup vs baseline: 1.0192x; 1.0192x over previous
import functools

import jax
import jax.numpy as jnp
from jax import lax
from jax.experimental import pallas as pl
from jax.experimental.pallas import tpu as pltpu
from jax.experimental.pallas import tpu_sc as plsc

_BATCH = 16384
_HIST = 50
_EMBED = 64
_B = _BATCH * _HIST

_info = plsc.get_sparse_core_info()
_NC, _NS = _info.num_cores, _info.num_subcores
_NW = _NC * _NS
_BPW = _B // _NW
_CH = 128
_NCHUNK = _BPW // _CH
_NB = 8
_K = _NB // 2

_mesh = plsc.VectorSubcoreMesh(core_axis_name="c", subcore_axis_name="s")


@functools.partial(
    pl.kernel,
    mesh=_mesh,
    out_type=jax.ShapeDtypeStruct((_B, _EMBED), jnp.float32),
    scratch_types=[
        pltpu.VMEM((_BPW,), jnp.int32),
    ]
    + [pltpu.VMEM((_CH, _EMBED), jnp.float32) for _ in range(_NB)]
    + [pltpu.SemaphoreType.DMA for _ in range(2 * _NB)],
    compiler_params=pltpu.CompilerParams(use_tc_tiling_on_sc=False),
)
def _gather_sc(idx_hbm, table_hbm, out_hbm, idx_v, *bufs_and_sems):
    rows = bufs_and_sems[:_NB]
    gsems = bufs_and_sems[_NB : 2 * _NB]
    ssems = bufs_and_sems[2 * _NB :]
    wid = lax.axis_index("s") * _NC + lax.axis_index("c")
    f0 = wid * _BPW
    pltpu.sync_copy(idx_hbm.at[wid], idx_v)

    def gather_start(c, b):
        pltpu.async_copy(
            table_hbm.at[idx_v.at[pl.ds(c * _CH, _CH)]], rows[b], gsems[b]
        )

    def gather_wait(c, b):
        pltpu.make_async_copy(
            table_hbm.at[idx_v.at[pl.ds(c * _CH, _CH)]], rows[b], gsems[b]
        ).wait()

    def store_start(c, b):
        pltpu.async_copy(
            rows[b], out_hbm.at[pl.ds(f0 + c * _CH, _CH)], ssems[b]
        )

    def store_wait(c, b):
        pltpu.make_async_copy(
            rows[b], out_hbm.at[pl.ds(f0 + c * _CH, _CH)], ssems[b]
        ).wait()

    for b in range(_K):
        gather_start(b, b)

    def body(c0):
        for b in range(_NB):
            c = c0 + b
            gather_wait(c, b)
            store_start(c, b)

            bk = (b - _K) % _NB

            @pl.when(c >= _K)
            def _():
                store_wait(c - _K, bk)

            @pl.when(c + _K < _NCHUNK)
            def _():
                gather_start(c + _K, bk)

    pl.loop(0, _NCHUNK, step=_NB)(body)

    for c in range(_NCHUNK - _K, _NCHUNK):
        store_wait(c, c % _NB)


def kernel(indices, table):
    idx = indices.astype(jnp.int32).reshape(_NW, _BPW)
    out = _gather_sc(idx, table)
    return out.reshape(_BATCH, _HIST, _EMBED)
